# Initial kernel scaffold; baseline (speedup 1.0000x reference)
import jax
import jax.numpy as jnp
from jax.experimental import pallas as pl


def kernel(motifs, reward, embedding_matrix):
    raise NotImplementedError("write your pallas kernel here")


# speedup vs baseline: 1.2802x; 1.2802x over previous
import functools

import jax
import jax.numpy as jnp
from jax import lax
from jax.experimental import pallas as pl
from jax.experimental.pallas import tpu as pltpu
from jax.experimental.pallas import tpu_sc as plsc

NC = 2
NS = 16
L = 16
NW = NC * NS

CHUNK = 128
GROUPS = CHUNK // L


def _tec_body(b, m, motifs_hbm, reward_hbm, table_hbm, out_hbm,
              raw_idx_v, idx_v, rows_v, rew_v, acc_v, sem, rsem):
    _, d = table_hbm.shape
    b_per_w = b // NW
    nchunk = b_per_w // CHUNK

    wid = lax.axis_index("s") * NC + lax.axis_index("c")
    iota = lax.iota(jnp.int32, L)

    def chunk_body(t, loss_acc):
        base = wid * b_per_w + t * CHUNK
        pltpu.sync_copy(motifs_hbm.at[pl.ds(base * m, CHUNK * m)], raw_idx_v)
        pltpu.async_copy(reward_hbm.at[pl.ds(base, CHUNK)], rew_v, rsem).wait()
        for mm in range(m):
            for j in range(GROUPS):
                v = plsc.load_gather(raw_idx_v, [(j * L + iota) * m + mm])
                idx_v[mm, pl.ds(j * L, L)] = v
        handles = [
            pltpu.async_copy(table_hbm.at[idx_v.at[mm]], rows_v.at[mm], sem)
            for mm in range(m)
        ]
        for h in handles:
            h.wait()

        def group_body(g, loss_acc):
            row = g * L + iota

            def dim_body(i, score):
                d0 = i * 8
                for k in range(8):
                    dv = jnp.full((L,), d0 + k, jnp.int32)
                    v0 = plsc.load_gather(rows_v, [jnp.zeros((L,), jnp.int32), row, dv])
                    v1 = plsc.load_gather(rows_v, [jnp.ones((L,), jnp.int32), row, dv])
                    v2 = plsc.load_gather(rows_v, [jnp.full((L,), 2, jnp.int32), row, dv])
                    score = score + v0 * v1 * v2
                return score

            score = lax.fori_loop(0, d // 8, dim_body, jnp.zeros((L,), jnp.float32))
            p = jnp.clip(1.0 - jnp.exp(-score), 1e-5, 1.0)
            rew = rew_v[pl.ds(g * L, L)]
            return loss_acc + p * rew

        return lax.fori_loop(0, GROUPS, group_body, loss_acc)

    loss_acc = lax.fori_loop(0, nchunk, chunk_body, jnp.zeros((L,), jnp.float32))
    acc_v[...] = loss_acc
    pltpu.sync_copy(acc_v, out_hbm.at[wid])


@functools.partial(jax.jit, static_argnames=())
def _run_sc(motifs, reward, embedding_matrix):
    b, m = motifs.shape
    _, d = embedding_matrix.shape
    mesh = plsc.VectorSubcoreMesh(core_axis_name="c", subcore_axis_name="s")
    kern = functools.partial(
        pl.kernel,
        mesh=mesh,
        compiler_params=pltpu.CompilerParams(needs_layout_passes=False),
        out_type=jax.ShapeDtypeStruct((NW, L), jnp.float32),
        scratch_types=[
            pltpu.VMEM((CHUNK * m,), jnp.int32),
            pltpu.VMEM((m, CHUNK), jnp.int32),
            pltpu.VMEM((m, CHUNK, d), jnp.float32),
            pltpu.VMEM((CHUNK,), jnp.float32),
            pltpu.VMEM((L,), jnp.float32),
            pltpu.SemaphoreType.DMA,
            pltpu.SemaphoreType.DMA,
        ],
    )(functools.partial(_tec_body, b, m))
    return kern(motifs.reshape(-1), reward, embedding_matrix)


def kernel(motifs, reward, embedding_matrix):
    partials = _run_sc(motifs, reward, embedding_matrix)
    return -(jnp.sum(partials) / motifs.shape[0])


# speedup vs baseline: 3.3820x; 2.6418x over previous
import functools

import jax
import jax.numpy as jnp
from jax import lax
from jax.experimental import pallas as pl
from jax.experimental.pallas import tpu as pltpu
from jax.experimental.pallas import tpu_sc as plsc

NC = 2
NS = 16
L = 16
NW = NC * NS

CHUNK = 128
GROUPS = CHUNK // L


def _tec_body(b, m, motifs_hbm, reward_hbm, table_hbm, out_hbm,
              raw_v, idx_v, rows0_v, rows1_v, rew_v, part_v, acc_v,
              sem_in, sem0, sem1):
    _, d = table_hbm.shape
    b_per_w = b // NW
    nchunk = b_per_w // CHUNK

    wid = lax.axis_index("s") * NC + lax.axis_index("c")
    iota = lax.iota(jnp.int32, L)

    pltpu.sync_copy(motifs_hbm.at[pl.ds(wid * b_per_w * m, b_per_w * m)], raw_v)
    pltpu.async_copy(reward_hbm.at[pl.ds(wid * b_per_w, b_per_w)], rew_v,
                     sem_in).wait()
    iota_m = iota * m
    for t in range(nchunk):
        for mm in range(m):
            for j in range(GROUPS):
                addr = iota_m + ((t * CHUNK + j * L) * m + mm)
                idx_v[t * m + mm, pl.ds(j * L, L)] = plsc.load_gather(raw_v, [addr])

    rows_bufs = (rows0_v, rows1_v)
    sems = (sem0, sem1)

    def fire(t):
        buf, sem = rows_bufs[t % 2], sems[t % 2]
        return [
            pltpu.async_copy(table_hbm.at[idx_v.at[t * m + mm]], buf.at[mm], sem)
            for mm in range(m)
        ]

    handles = fire(0)
    loss_acc = jnp.zeros((L,), jnp.float32)
    for t in range(nchunk):
        nxt = fire(t + 1) if t + 1 < nchunk else None
        for h in handles:
            h.wait()
        buf = rows_bufs[t % 2]

        def elem_body(e, carry, buf=buf):
            acc = jnp.zeros((L,), jnp.float32)
            for c in range(d // L):
                sl = pl.ds(c * L, L)
                prod = buf[0, e, sl]
                for mm in range(1, m):
                    prod = prod * buf[mm, e, sl]
                acc = acc + prod
            part_v[pl.ds(e * L, L)] = acc
            return carry
        lax.fori_loop(0, CHUNK, elem_body, 0, unroll=2)

        def group_body(g, loss_acc, t=t):
            rowbase = (g * L + iota) * L
            score = jnp.zeros((L,), jnp.float32)
            for j in range(L):
                score = score + plsc.load_gather(part_v, [rowbase + j])
            p = jnp.clip(1.0 - jnp.exp(-score), 1e-5, 1.0)
            rew = rew_v[pl.ds(t * CHUNK + g * L, L)]
            return loss_acc + p * rew
        loss_acc = lax.fori_loop(0, GROUPS, group_body, loss_acc)
        handles = nxt

    acc_v[...] = loss_acc
    pltpu.sync_copy(acc_v, out_hbm.at[wid])


@jax.jit
def _run_sc(motifs, reward, embedding_matrix):
    b, m = motifs.shape
    _, d = embedding_matrix.shape
    b_per_w = b // NW
    nchunk = b_per_w // CHUNK
    mesh = plsc.VectorSubcoreMesh(core_axis_name="c", subcore_axis_name="s")
    kern = functools.partial(
        pl.kernel,
        mesh=mesh,
        compiler_params=pltpu.CompilerParams(needs_layout_passes=False),
        out_type=jax.ShapeDtypeStruct((NW, L), jnp.float32),
        scratch_types=[
            pltpu.VMEM((b_per_w * m,), jnp.int32),
            pltpu.VMEM((nchunk * m, CHUNK), jnp.int32),
            pltpu.VMEM((m, CHUNK, d), jnp.float32),
            pltpu.VMEM((m, CHUNK, d), jnp.float32),
            pltpu.VMEM((b_per_w,), jnp.float32),
            pltpu.VMEM((CHUNK * L,), jnp.float32),
            pltpu.VMEM((L,), jnp.float32),
            pltpu.SemaphoreType.DMA,
            pltpu.SemaphoreType.DMA,
            pltpu.SemaphoreType.DMA,
        ],
    )(functools.partial(_tec_body, b, m))
    return kern(motifs.reshape(-1), reward, embedding_matrix)


def kernel(motifs, reward, embedding_matrix):
    partials = _run_sc(motifs, reward, embedding_matrix)
    return -(jnp.sum(partials) / motifs.shape[0])
